# baseline (speedup 1.0000x reference)
import functools

import jax
import jax.numpy as jnp
from jax import lax
from jax.experimental import pallas as pl
from jax.experimental.pallas import tpu as pltpu
from jax.experimental.pallas import tpu_sc as plsc

_DEPTH_MIN = 1.0
_DEPTH_MAX = 57.0
_DEPTH_CHANNELS = 112

_B, _C, _H, _W = 24, 112, 32, 88
_PIX = _H * _W
_N = _B * _PIX
_NC, _NS, _L = 2, 16, 16
_NW = _NC * _NS
_PER_W = _N // _NW
_ROW = 96
_NROW = _PER_W // _ROW
_GRP = _ROW // _L

_LN2 = 0.6931471805599453
_SQRT2 = 1.4142135623730951


def _log_f32(x):
    bits = lax.bitcast_convert_type(x, jnp.int32)
    e = (bits >> 23) - 127
    m = lax.bitcast_convert_type(
        (bits & 0x007FFFFF) | 0x3F800000, jnp.float32)
    big = m > _SQRT2
    m = jnp.where(big, m * 0.5, m)
    e_f = e.astype(jnp.float32) + jnp.where(big, 1.0, 0.0)
    t = (m - 1.0) / (m + 1.0)
    t2 = t * t
    p = 2.0 + t2 * (2.0 / 3.0 + t2 * (0.4 + t2 * (2.0 / 7.0 + t2 * (2.0 / 9.0))))
    return e_f * _LN2 + t * p


def _sc_body(pred_hbm, gt_hbm, vm_hbm, loss_hbm, vmsum_hbm,
             gt_v, vm_v, idx_v, gat_v, stage_v, sem):
    wid = lax.axis_index("s") * _NC + lax.axis_index("c")
    base = wid * _PER_W

    pltpu.sync_copy(gt_hbm.at[pl.ds(base, _PER_W)], gt_v)
    pltpu.sync_copy(vm_hbm.at[pl.ds(base, _PER_W)], vm_v)

    lane = lax.iota(jnp.int32, _L)

    def idx_row(r, _):
        for c in range(_GRP):
            off = r * _ROW + c * _L
            g = gt_v[pl.ds(off, _L)]
            bin_ = jnp.minimum(jnp.maximum(2 * g - 2, 0), _DEPTH_CHANNELS - 1)
            p = base + off + lane
            img = (p.astype(jnp.float32) * (1.0 / _PIX)).astype(jnp.int32)
            img = jnp.where(img * _PIX > p, img - 1, img)
            img = jnp.where((img + 1) * _PIX <= p, img + 1, img)
            pix = p - img * _PIX
            idx_v[r, pl.ds(c * _L, _L)] = img * (_C * _PIX) + bin_ * _PIX + pix
        return 0

    lax.fori_loop(0, _NROW, idx_row, 0)

    copies = [
        pltpu.make_async_copy(pred_hbm.at[idx_v.at[r]], gat_v.at[r], sem)
        for r in range(_NROW)
    ]
    for cp in copies:
        cp.start()
    for cp in copies:
        cp.wait()

    def acc_row(r, carry):
        ls, vs = carry
        for c in range(_GRP):
            off = r * _ROW + c * _L
            x = gat_v[r, pl.ds(c * _L, _L)] + 1e-8
            vm = vm_v[pl.ds(off, _L)]
            ls = ls + vm * _log_f32(x)
            vs = vs + vm
        return ls, vs

    zero = jnp.zeros((_L,), jnp.float32)
    ls, vs = lax.fori_loop(0, _NROW, acc_row, (zero, zero))

    stage_v[pl.ds(0, _L)] = ls
    pltpu.sync_copy(stage_v, loss_hbm.at[wid])
    stage_v[pl.ds(0, _L)] = vs
    pltpu.sync_copy(stage_v, vmsum_hbm.at[wid])


_sc_call = pl.kernel(
    _sc_body,
    out_type=(
        jax.ShapeDtypeStruct((_NW, _L), jnp.float32),
        jax.ShapeDtypeStruct((_NW, _L), jnp.float32),
    ),
    mesh=plsc.VectorSubcoreMesh(
        core_axis_name="c", subcore_axis_name="s",
        num_cores=_NC, num_subcores=_NS),
    scratch_types=[
        pltpu.VMEM((_PER_W,), jnp.int32),
        pltpu.VMEM((_PER_W,), jnp.float32),
        pltpu.VMEM((_NROW, _ROW), jnp.int32),
        pltpu.VMEM((_NROW, _ROW), jnp.float32),
        pltpu.VMEM((_L,), jnp.float32),
        pltpu.SemaphoreType.DMA,
    ],
)


@jax.jit
def kernel(pred_depth, gt_depth_map, valid_mask):
    loss_part, vm_part = _sc_call(
        pred_depth.reshape(-1),
        gt_depth_map.reshape(-1).astype(jnp.int32),
        valid_mask.reshape(-1),
    )
    neg_wsum = jnp.sum(loss_part)
    vm_sum = jnp.sum(vm_part)
    weighted = -neg_wsum / jnp.maximum(vm_sum, 1e-12)
    return jnp.where(vm_sum > 0, weighted, jnp.float32(0.0))


# speedup vs baseline: 1.5966x; 1.5966x over previous
import jax
import jax.numpy as jnp
from jax import lax
from jax.experimental import pallas as pl
from jax.experimental.pallas import tpu as pltpu
from jax.experimental.pallas import tpu_sc as plsc

_B, _C, _H, _W = 24, 112, 32, 88
_NC, _NS, _L = 2, 16, 16
_NW = _NC * _NS
_OCT = 8
_NOCT = _B * _H // _OCT
_PER_W = _NOCT // _NW
_ROWS_W = _PER_W * _OCT
_NBIN = 56
_G0 = (0, 16, 32, 48, 64, 72)

_LN2 = 0.6931471805599453
_SQRT2 = 1.4142135623730951


def _log_f32(x):
    bits = lax.bitcast_convert_type(x, jnp.int32)
    e = (bits >> 23) - 127
    m = lax.bitcast_convert_type(
        (bits & 0x007FFFFF) | 0x3F800000, jnp.float32)
    big = m > _SQRT2
    m = jnp.where(big, m * 0.5, m)
    e_f = e.astype(jnp.float32) + jnp.where(big, 1.0, 0.0)
    t = (m - 1.0) / (m + 1.0)
    t2 = t * t
    p = 2.0 + t2 * (2.0 / 3.0 + t2 * (0.4 + t2 * (2.0 / 7.0 + t2 * (2.0 / 9.0))))
    return e_f * _LN2 + t * p


def _sc_body(pred_hbm, gt_hbm, vm_hbm, out_hbm,
             gt_v, vm_v, blk_a, blk_b, stage_v, sem_a, sem_b):
    wid = lax.axis_index("s") * _NC + lax.axis_index("c")
    rowbase = wid * _ROWS_W

    pltpu.sync_copy(gt_hbm.at[pl.ds(rowbase, _ROWS_W)], gt_v)
    pltpu.sync_copy(vm_hbm.at[pl.ds(rowbase, _ROWS_W)], vm_v)

    lane = lax.iota(jnp.int32, _L)
    bufs = (blk_a, blk_b)
    sems = (sem_a, sem_b)

    def _copy(j):
        go = wid * _PER_W + j
        return pltpu.make_async_copy(
            pred_hbm.at[go >> 2, :, 0, go & 3], bufs[j % 2], sems[j % 2])

    _copy(0).start()
    _copy(1).start()

    zero = jnp.zeros((_L,), jnp.float32)
    ls, vs = zero, zero
    for j in range(_PER_W):
        _copy(j).wait()
        blk = bufs[j % 2]
        for hi in range(_OCT):
            g_row = _OCT * j + hi
            hi_vec = jnp.zeros((_L,), jnp.int32) + hi
            for gi, w0 in enumerate(_G0):
                cols = w0 + lane
                g = gt_v[g_row, pl.ds(w0, _L)]
                slot = jnp.minimum(jnp.maximum(g - 1, 0), _NBIN - 1)
                x = plsc.load_gather(blk, [slot, hi_vec, cols]) + 1e-8
                vm = vm_v[g_row, pl.ds(w0, _L)]
                if gi == len(_G0) - 1:
                    vm = jnp.where(lane >= 8, vm, 0.0)
                ls = ls + vm * _log_f32(x)
                vs = vs + vm
        if j + 2 < _PER_W:
            _copy(j + 2).start()

    stage_v[pl.ds(0, _L)] = ls
    pltpu.sync_copy(stage_v, out_hbm.at[0, wid])
    stage_v[pl.ds(0, _L)] = vs
    pltpu.sync_copy(stage_v, out_hbm.at[1, wid])


_sc_call = pl.kernel(
    _sc_body,
    out_type=jax.ShapeDtypeStruct((2, _NW, _L), jnp.float32),
    mesh=plsc.VectorSubcoreMesh(
        core_axis_name="c", subcore_axis_name="s",
        num_cores=_NC, num_subcores=_NS),
    compiler_params=pltpu.CompilerParams(needs_layout_passes=False),
    scratch_types=[
        pltpu.VMEM((_ROWS_W, _W), jnp.int32),
        pltpu.VMEM((_ROWS_W, _W), jnp.float32),
        pltpu.VMEM((_NBIN, _OCT, _W), jnp.float32),
        pltpu.VMEM((_NBIN, _OCT, _W), jnp.float32),
        pltpu.VMEM((_L,), jnp.float32),
        pltpu.SemaphoreType.DMA,
        pltpu.SemaphoreType.DMA,
    ],
)


@jax.jit
def kernel(pred_depth, gt_depth_map, valid_mask):
    parts = _sc_call(
        pred_depth.reshape(_B, _NBIN, 2, _H // _OCT, _OCT, _W),
        gt_depth_map.reshape(_B * _H, _W).astype(jnp.int32),
        valid_mask.reshape(_B * _H, _W),
    )
    sums = jnp.sum(parts, axis=(1, 2))
    neg_wsum, vm_sum = sums[0], sums[1]
    weighted = -neg_wsum / jnp.maximum(vm_sum, 1e-12)
    return jnp.where(vm_sum > 0, weighted, jnp.float32(0.0))


# speedup vs baseline: 2.5838x; 1.6183x over previous
import jax
import jax.numpy as jnp
from jax import lax
from jax.experimental import pallas as pl
from jax.experimental.pallas import tpu as pltpu
from jax.experimental.pallas import tpu_sc as plsc

_B, _C, _H, _W = 24, 112, 32, 88
_NPIX = _B * _H * _W
_NC, _NS, _L = 2, 16, 16
_NW = _NC * _NS
_PIX_W = _NPIX // _NW
_ROWS_W = _PIX_W // _W
_CHUNK_ROWS = 4
_CHUNK = _CHUNK_ROWS * _W
_NCHUNK = _PIX_W // _CHUNK
_G0 = (0, 16, 32, 48, 64, 72)

_LN2 = 0.6931471805599453
_SQRT2 = 1.4142135623730951


def _log_f32(x):
    bits = lax.bitcast_convert_type(x, jnp.int32)
    e = (bits >> 23) - 127
    m = lax.bitcast_convert_type(
        (bits & 0x007FFFFF) | 0x3F800000, jnp.float32)
    big = m > _SQRT2
    m = jnp.where(big, m * 0.5, m)
    e_f = e.astype(jnp.float32) + jnp.where(big, 1.0, 0.0)
    t = (m - 1.0) / (m + 1.0)
    t2 = t * t
    p = 2.0 + t2 * (2.0 / 3.0 + t2 * (0.4 + t2 * (2.0 / 7.0 + t2 * (2.0 / 9.0))))
    return e_f * _LN2 + t * p


def _sc_body(pred_hbm, gt_hbm, vm_hbm, out_hbm,
             gt_v, vm_v, blk_a, blk_b, stage_v, sem_a, sem_b):
    wid = lax.axis_index("s") * _NC + lax.axis_index("c")
    rowbase = wid * _ROWS_W
    pixbase = wid * _PIX_W

    pltpu.sync_copy(gt_hbm.at[pl.ds(rowbase, _ROWS_W)], gt_v)
    pltpu.sync_copy(vm_hbm.at[pl.ds(rowbase, _ROWS_W)], vm_v)

    lane = lax.iota(jnp.int32, _L)
    bufs = (blk_a, blk_b)
    sems = (sem_a, sem_b)

    def _copy(j):
        return pltpu.make_async_copy(
            pred_hbm.at[pl.ds(pixbase + j * _CHUNK, _CHUNK)],
            bufs[j % 2], sems[j % 2])

    _copy(0).start()
    _copy(1).start()

    zero = jnp.zeros((_L,), jnp.float32)
    ls, vs = zero, zero
    for j in range(_NCHUNK):
        _copy(j).wait()
        blk = bufs[j % 2]
        for ri in range(_CHUNK_ROWS):
            g_row = _CHUNK_ROWS * j + ri
            for gi, w0 in enumerate(_G0):
                p_loc = ri * _W + w0 + lane
                g = gt_v[g_row, pl.ds(w0, _L)]
                bin_ = jnp.minimum(jnp.maximum(2 * g - 2, 0), _C - 1)
                x = plsc.load_gather(blk, [p_loc, bin_]) + 1e-8
                vm = vm_v[g_row, pl.ds(w0, _L)]
                if gi == len(_G0) - 1:
                    vm = jnp.where(lane >= 8, vm, 0.0)
                ls = ls + vm * _log_f32(x)
                vs = vs + vm
        if j + 2 < _NCHUNK:
            _copy(j + 2).start()

    stage_v[pl.ds(0, _L)] = ls
    pltpu.sync_copy(stage_v, out_hbm.at[0, wid])
    stage_v[pl.ds(0, _L)] = vs
    pltpu.sync_copy(stage_v, out_hbm.at[1, wid])


_sc_call = pl.kernel(
    _sc_body,
    out_type=jax.ShapeDtypeStruct((2, _NW, _L), jnp.float32),
    mesh=plsc.VectorSubcoreMesh(
        core_axis_name="c", subcore_axis_name="s",
        num_cores=_NC, num_subcores=_NS),
    compiler_params=pltpu.CompilerParams(needs_layout_passes=False),
    scratch_types=[
        pltpu.VMEM((_ROWS_W, _W), jnp.int32),
        pltpu.VMEM((_ROWS_W, _W), jnp.float32),
        pltpu.VMEM((_CHUNK, _C), jnp.float32),
        pltpu.VMEM((_CHUNK, _C), jnp.float32),
        pltpu.VMEM((_L,), jnp.float32),
        pltpu.SemaphoreType.DMA,
        pltpu.SemaphoreType.DMA,
    ],
)


@jax.jit
def kernel(pred_depth, gt_depth_map, valid_mask):
    parts = _sc_call(
        pred_depth.transpose(0, 2, 3, 1).reshape(_NPIX, _C),
        gt_depth_map.reshape(_B * _H, _W).astype(jnp.int32),
        valid_mask.reshape(_B * _H, _W),
    )
    sums = jnp.sum(parts, axis=(1, 2))
    neg_wsum, vm_sum = sums[0], sums[1]
    weighted = -neg_wsum / jnp.maximum(vm_sum, 1e-12)
    return jnp.where(vm_sum > 0, weighted, jnp.float32(0.0))


# speedup vs baseline: 2.8060x; 1.0860x over previous
import jax
import jax.numpy as jnp
from jax import lax
from jax.experimental import pallas as pl
from jax.experimental.pallas import tpu as pltpu
from jax.experimental.pallas import tpu_sc as plsc

_B, _C, _H, _W = 24, 112, 32, 88
_NPIX = _B * _H * _W
_NC, _NS, _L = 2, 16, 16
_NW = _NC * _NS
_PIX_W = _NPIX // _NW
_ROWS_W = _PIX_W // _W
_CHUNK_ROWS = 4
_CHUNK = _CHUNK_ROWS * _W
_NCHUNK = _PIX_W // _CHUNK
_G0 = (0, 16, 32, 48, 64, 72)

_LN2 = 0.6931471805599453
_SQRT2 = 1.4142135623730951


def _log_f32(x):
    bits = lax.bitcast_convert_type(x, jnp.int32)
    e = (bits >> 23) - 127
    m = lax.bitcast_convert_type(
        (bits & 0x007FFFFF) | 0x3F800000, jnp.float32)
    big = m > _SQRT2
    m = jnp.where(big, m * 0.5, m)
    e_f = e.astype(jnp.float32) + jnp.where(big, 1.0, 0.0)
    t = (m - 1.0) / (m + 1.0)
    t2 = t * t
    p = 2.0 + t2 * (2.0 / 3.0 + t2 * (0.4 + t2 * (2.0 / 7.0 + t2 * (2.0 / 9.0))))
    return e_f * _LN2 + t * p


def _sc_body(pred_hbm, gt_hbm, vm_hbm, out_hbm,
             gt_v, vm_v, blk_a, blk_b, stage_v, sem_a, sem_b):
    wid = lax.axis_index("s") * _NC + lax.axis_index("c")
    rowbase = wid * _ROWS_W
    pixbase = wid * _PIX_W

    pltpu.sync_copy(gt_hbm.at[pl.ds(rowbase, _ROWS_W)], gt_v)
    pltpu.sync_copy(vm_hbm.at[pl.ds(rowbase, _ROWS_W)], vm_v)

    lane = lax.iota(jnp.int32, _L)
    bufs = (blk_a, blk_b)
    sems = (sem_a, sem_b)

    def _copy(j, b):
        return pltpu.make_async_copy(
            pred_hbm.at[pl.ds(pixbase + j * _CHUNK, _CHUNK)],
            bufs[b], sems[b])

    _copy(0, 0).start()
    _copy(1, 1).start()

    zero = jnp.zeros((_L,), jnp.float32)

    def _pair(p, carry):
        ls, vs = carry
        for b in range(2):
            j = 2 * p + b
            _copy(j, b).wait()

            def _row(ri, carry):
                ls, vs = carry
                g_row = j * _CHUNK_ROWS + ri
                for gi, w0 in enumerate(_G0):
                    p_loc = ri * _W + w0 + lane
                    g = gt_v[g_row, pl.ds(w0, _L)]
                    bin_ = jnp.minimum(jnp.maximum(2 * g - 2, 0), _C - 1)
                    x = plsc.load_gather(bufs[b], [p_loc, bin_]) + 1e-8
                    vm = vm_v[g_row, pl.ds(w0, _L)]
                    if gi == len(_G0) - 1:
                        vm = jnp.where(lane >= 8, vm, 0.0)
                    ls = ls + vm * _log_f32(x)
                    vs = vs + vm
                return ls, vs

            ls, vs = lax.fori_loop(0, _CHUNK_ROWS, _row, (ls, vs))

            @pl.when(j + 2 < _NCHUNK)
            def _():
                _copy(j + 2, b).start()
        return ls, vs

    ls, vs = lax.fori_loop(0, _NCHUNK // 2, _pair, (zero, zero))

    stage_v[pl.ds(0, _L)] = ls
    pltpu.sync_copy(stage_v, out_hbm.at[0, wid])
    stage_v[pl.ds(0, _L)] = vs
    pltpu.sync_copy(stage_v, out_hbm.at[1, wid])


_sc_call = pl.kernel(
    _sc_body,
    out_type=jax.ShapeDtypeStruct((2, _NW, _L), jnp.float32),
    mesh=plsc.VectorSubcoreMesh(
        core_axis_name="c", subcore_axis_name="s",
        num_cores=_NC, num_subcores=_NS),
    compiler_params=pltpu.CompilerParams(needs_layout_passes=False),
    scratch_types=[
        pltpu.VMEM((_ROWS_W, _W), jnp.int32),
        pltpu.VMEM((_ROWS_W, _W), jnp.float32),
        pltpu.VMEM((_CHUNK, _C), jnp.float32),
        pltpu.VMEM((_CHUNK, _C), jnp.float32),
        pltpu.VMEM((_L,), jnp.float32),
        pltpu.SemaphoreType.DMA,
        pltpu.SemaphoreType.DMA,
    ],
)


@jax.jit
def kernel(pred_depth, gt_depth_map, valid_mask):
    parts = _sc_call(
        pred_depth.transpose(0, 2, 3, 1).reshape(_NPIX, _C),
        gt_depth_map.reshape(_B * _H, _W).astype(jnp.int32),
        valid_mask.reshape(_B * _H, _W),
    )
    sums = jnp.sum(parts, axis=(1, 2))
    neg_wsum, vm_sum = sums[0], sums[1]
    weighted = -neg_wsum / jnp.maximum(vm_sum, 1e-12)
    return jnp.where(vm_sum > 0, weighted, jnp.float32(0.0))
